# baseline (speedup 1.0000x reference)
import functools

import jax
import jax.numpy as jnp
from jax import lax
from jax.experimental import pallas as pl
from jax.experimental.pallas import tpu as pltpu
from jax.experimental.pallas import tpu_sc as plsc

NC = 2
NS = 16
L = 16
K = 128


def _node_pad(n):
    return -(-(n + 1) // (NS * K)) * (NS * K)


def _deg_call(dst_pad, n):
    e_pad = dst_pad.shape[0]
    nw = NC * NS
    epw = e_pad // nw
    n_pad = _node_pad(n)
    nrows = n_pad // L
    spt = nrows // NS
    nrch = nrows // K
    mesh = plsc.VectorSubcoreMesh(core_axis_name="c", subcore_axis_name="s")
    io = jnp.arange(nrows, dtype=jnp.int32).reshape(nrch, K)

    @functools.partial(
        pl.kernel,
        mesh=mesh,
        out_type=jax.ShapeDtypeStruct((NC, nrows, L), jnp.float32),
        scratch_types=[
            pltpu.VMEM((epw,), jnp.int32),
            pltpu.VMEM((nrows, L), jnp.float32),
            pltpu.VMEM((nrch, K), jnp.int32),
            pltpu.VMEM_SHARED((nrows, L), jnp.float32),
        ],
        compiler_params=pltpu.CompilerParams(needs_layout_passes=False),
    )
    def deg_kernel(dst_hbm, io_hbm, out_hbm, idx_v, deg_v, io_v, sh_deg):
        c = lax.axis_index("c")
        s = lax.axis_index("s")
        wid = c * NS + s
        zeros = jnp.zeros((L,), jnp.float32)

        def zbody(i, carry):
            deg_v[i, pl.ds(0, L)] = zeros
            return carry

        lax.fori_loop(0, nrows, zbody, 0)
        pltpu.sync_copy(deg_v.at[pl.ds(0, spt)],
                        sh_deg.at[pl.ds(s * spt, spt)])
        pltpu.sync_copy(io_hbm, io_v)
        pltpu.sync_copy(dst_hbm.at[pl.ds(wid * epw, epw)], idx_v)
        ones = jnp.ones((L,), jnp.float32)
        four = jnp.full((L,), 4, jnp.int32)
        fifteen = jnp.full((L,), L - 1, jnp.int32)
        plsc.subcore_barrier()

        def hbody(i, carry):
            idx = idx_v[pl.ds(i * L, L)]
            row = lax.shift_right_logical(idx, four)
            col = idx & fifteen
            plsc.addupdate_scatter(deg_v, [row, col], ones)
            return carry

        lax.fori_loop(0, epw // L, hbody, 0)
        for k in range(nrch):
            pltpu.sync_copy(deg_v.at[pl.ds(k * K, K)],
                            sh_deg.at[io_v.at[k]], add=True)
        plsc.subcore_barrier()
        pltpu.sync_copy(sh_deg.at[pl.ds(s * spt, spt)],
                        out_hbm.at[c, pl.ds(s * spt, spt)])

    return deg_kernel(dst_pad, io)


IDX_BITS = 14


def _scatter_call(hp, packed_pad):
    n, d = hp.shape
    e_pad = packed_pad.shape[0]
    nw = NC * NS
    epw = e_pad // nw
    nchunks = epw // K
    n_pad = _node_pad(n)
    zpt = n_pad // NS
    mesh = plsc.VectorSubcoreMesh(core_axis_name="c", subcore_axis_name="s")
    packed3 = packed_pad.reshape(nw, nchunks, K)

    @functools.partial(
        pl.kernel,
        mesh=mesh,
        out_type=jax.ShapeDtypeStruct((NC, n_pad, d), jnp.float32),
        scratch_types=(
            [pltpu.VMEM((nchunks, K), jnp.int32)]
            + [pltpu.VMEM((K,), jnp.int32)] * 4
            + [pltpu.VMEM((K, d), jnp.float32)] * 2
            + [pltpu.VMEM_SHARED((n_pad, d), jnp.float32)]
            + [pltpu.SemaphoreType.DMA] * 4
        ),
    )
    def scat_kernel(hp_hbm, pk_hbm, out_hbm, *scr):
        ptab = scr[0]
        sbuf = scr[1:3]
        dbuf = scr[3:5]
        bufs = scr[5:7]
        acc_sh = scr[7]
        gsem = scr[8:10]
        ssem = scr[10:12]
        c = lax.axis_index("c")
        s = lax.axis_index("s")
        wid = c * NS + s
        pltpu.sync_copy(pk_hbm.at[wid], ptab)
        zeros = jnp.zeros((L,), jnp.float32)
        cols = d // L

        def zbody(i, carry):
            for c2 in range(cols):
                bufs[0][i, pl.ds(c2 * L, L)] = zeros
            return carry

        lax.fori_loop(0, K, zbody, 0)
        full = n // zpt
        rem = n - full * zpt

        @pl.when(jnp.logical_and(c == 0, s < full))
        def _():
            pltpu.sync_copy(hp_hbm.at[pl.ds(s * zpt, zpt)],
                            acc_sh.at[pl.ds(s * zpt, zpt)])

        @pl.when(jnp.logical_and(c == 0, s == full))
        def _():
            if rem:
                pltpu.sync_copy(hp_hbm.at[pl.ds(s * zpt, rem)],
                                acc_sh.at[pl.ds(s * zpt, rem)])
            off = rem
            while off < zpt:
                piece = min(K, zpt - off)
                pltpu.sync_copy(bufs[0].at[pl.ds(0, piece)],
                                acc_sh.at[pl.ds(s * zpt + off, piece)])
                off += piece

        @pl.when(jnp.logical_or(c > 0, s > full))
        def _():
            for k2 in range(zpt // K):
                pltpu.sync_copy(bufs[0],
                                acc_sh.at[pl.ds(s * zpt + k2 * K, K)])

        mask = jnp.full((L,), (1 << IDX_BITS) - 1, jnp.int32)
        shift = jnp.full((L,), IDX_BITS, jnp.int32)

        def unpack(j, b):
            for t in range(K // L):
                p = ptab[j, pl.ds(t * L, L)]
                sbuf[b][pl.ds(t * L, L)] = p & mask
                dbuf[b][pl.ds(t * L, L)] = lax.shift_right_logical(p, shift)

        def gather_start(j, b):
            unpack(j, b)
            pltpu.async_copy(hp_hbm.at[sbuf[b]], bufs[b], gsem[b])

        def scatter_start(b):
            pltpu.async_copy(bufs[b], acc_sh.at[dbuf[b]], ssem[b], add=True)

        def wait_chunk(sem, b):
            pltpu.make_async_copy(hp_hbm.at[pl.ds(0, K)], bufs[b], sem).wait()

        plsc.subcore_barrier()

        def body(g, carry):
            for b in range(2):
                j = 2 * g + b

                @pl.when(g > 0)
                def _():
                    wait_chunk(ssem[b], b)
                gather_start(j, b)
                if b == 0:
                    @pl.when(g > 0)
                    def _():
                        wait_chunk(gsem[1], 1)
                        scatter_start(1)
                else:
                    wait_chunk(gsem[0], 0)
                    scatter_start(0)
            return carry

        lax.fori_loop(0, nchunks // 2, body, 0)
        wait_chunk(gsem[1], 1)
        scatter_start(1)
        wait_chunk(ssem[0], 0)
        wait_chunk(ssem[1], 1)
        plsc.subcore_barrier()
        pltpu.sync_copy(acc_sh.at[pl.ds(s * zpt, zpt)],
                        out_hbm.at[c, pl.ds(s * zpt, zpt)])

    return scat_kernel(hp, packed3)


def _dinv_cols(d0, d1):
    return lax.rsqrt(d0 + d1 + 1.0)


def _matmul_call(x, W):
    n, d = x.shape
    blk = 2000
    grid = n // blk

    def body(x_ref, w_ref, o_ref):
        o_ref[...] = jnp.dot(x_ref[...], w_ref[...],
                             preferred_element_type=jnp.float32)

    return pl.pallas_call(
        body,
        grid=(grid,),
        in_specs=[
            pl.BlockSpec((blk, d), lambda i: (i, 0)),
            pl.BlockSpec((d, d), lambda i: (0, 0)),
        ],
        out_specs=pl.BlockSpec((blk, d), lambda i: (i, 0)),
        out_shape=jax.ShapeDtypeStruct((n, d), jnp.float32),
    )(x, W)


def _scale_call(h, d0, d1):
    n, d = h.shape
    blk = 2000
    grid = n // blk

    def body(h_ref, d0_ref, d1_ref, o_ref):
        dinv = _dinv_cols(d0_ref[...], d1_ref[...])
        o_ref[...] = h_ref[...] * dinv

    return pl.pallas_call(
        body,
        grid=(grid,),
        in_specs=[
            pl.BlockSpec((blk, d), lambda i: (i, 0)),
            pl.BlockSpec((blk, 1), lambda i: (i, 0)),
            pl.BlockSpec((blk, 1), lambda i: (i, 0)),
        ],
        out_specs=pl.BlockSpec((blk, d), lambda i: (i, 0)),
        out_shape=jax.ShapeDtypeStruct((n, d), jnp.float32),
    )(h, d0, d1)


def _epilogue_call(x, s_parts, d0, d1, b2):
    n, d = x.shape
    blk = 2000
    grid = n // blk

    def body(x_ref, s0_ref, s1_ref, d0_ref, d1_ref, b_ref, o_ref):
        dinv = _dinv_cols(d0_ref[...], d1_ref[...])
        stot = s0_ref[...].reshape(blk, d) + s1_ref[...].reshape(blk, d)
        agg = stot * dinv + b_ref[...]
        o_ref[...] = x_ref[...] + jnp.maximum(agg, 0.0)

    return pl.pallas_call(
        body,
        grid=(grid,),
        in_specs=[
            pl.BlockSpec((blk, d), lambda i: (i, 0)),
            pl.BlockSpec((1, blk, d), lambda i: (0, i, 0)),
            pl.BlockSpec((1, blk, d), lambda i: (1, i, 0)),
            pl.BlockSpec((blk, 1), lambda i: (i, 0)),
            pl.BlockSpec((blk, 1), lambda i: (i, 0)),
            pl.BlockSpec((1, d), lambda i: (0, 0)),
        ],
        out_specs=pl.BlockSpec((blk, d), lambda i: (i, 0)),
        out_shape=jax.ShapeDtypeStruct((n, d), jnp.float32),
    )(x, s_parts, s_parts, d0, d1, b2)


def kernel(x, edge_index, W, b):
    n, d = x.shape
    e = edge_index.shape[1]
    cpt = NC * NS * K * 2
    e_pad = -(-e // cpt) * cpt
    src = edge_index[0]
    dst = edge_index[1]
    pad = e_pad - e
    if pad:
        pi = jnp.arange(pad, dtype=jnp.int32)
        src = jnp.concatenate([src, pi % n])
        dst = jnp.concatenate([dst, n + pi % (_node_pad(n) - n)])
    packed = jnp.bitwise_or(jnp.left_shift(dst, IDX_BITS), src)

    n_pad = _node_pad(n)
    deg_sc = _deg_call(dst, n).reshape(NC, n_pad)
    d0 = deg_sc[0].reshape(n_pad, 1)
    d1 = deg_sc[1].reshape(n_pad, 1)
    h = _matmul_call(x, W)
    hp = _scale_call(h, d0, d1)
    s_parts = _scatter_call(hp, packed)
    y = _epilogue_call(x, s_parts, d0, d1, b.reshape(1, d))
    return y


# speedup vs baseline: 1.0769x; 1.0769x over previous
import functools

import numpy as np

import jax
import jax.numpy as jnp
from jax import lax
from jax.experimental import pallas as pl
from jax.experimental.pallas import tpu as pltpu
from jax.experimental.pallas import tpu_sc as plsc

NC = 2
NS = 16
L = 16
K = 128


def _node_pad(n):
    return -(-(n + 1) // (NS * K)) * (NS * K)


def _deg_call(dst_pad, n):
    e_pad = dst_pad.shape[0]
    nw = NC * NS
    epw = e_pad // nw
    n_pad = _node_pad(n)
    mesh = plsc.VectorSubcoreMesh(core_axis_name="c", subcore_axis_name="s")

    @functools.partial(
        pl.kernel,
        mesh=mesh,
        out_type=jax.ShapeDtypeStruct((nw, n_pad), jnp.float32),
        scratch_types=[
            pltpu.VMEM((epw,), jnp.int32),
            pltpu.VMEM((n_pad,), jnp.float32),
        ],
        compiler_params=pltpu.CompilerParams(needs_layout_passes=False),
    )
    def deg_kernel(pk_hbm, out_hbm, idx_v, deg_v):
        c = lax.axis_index("c")
        s = lax.axis_index("s")
        wid = c * NS + s
        zeros = jnp.zeros((L,), jnp.float32)

        def zbody(i, carry):
            deg_v[pl.ds(i * L, L)] = zeros
            return carry

        lax.fori_loop(0, n_pad // L, zbody, 0)
        pltpu.sync_copy(pk_hbm.at[pl.ds(wid * epw, epw)], idx_v)
        ones = jnp.ones((L,), jnp.float32)
        shift = jnp.full((L,), IDX_BITS, jnp.int32)

        def hbody(i, carry):
            idx = lax.shift_right_logical(idx_v[pl.ds(i * L, L)], shift)
            plsc.addupdate_scatter(deg_v, [idx], ones)
            return carry

        lax.fori_loop(0, epw // L, hbody, 0)
        pltpu.sync_copy(deg_v, out_hbm.at[wid])

    return deg_kernel(dst_pad)


IDX_BITS = 14


def _scatter_call(hp, packed_pad):
    n, d = hp.shape
    e_pad = packed_pad.shape[0]
    nw = NC * NS
    epw = e_pad // nw
    nchunks = epw // K
    n_pad = _node_pad(n)
    zpt = n_pad // NS
    mesh = plsc.VectorSubcoreMesh(core_axis_name="c", subcore_axis_name="s")
    packed3 = packed_pad.reshape(nw, nchunks, K)

    @functools.partial(
        pl.kernel,
        mesh=mesh,
        out_type=jax.ShapeDtypeStruct((NC, n_pad, d), jnp.float32),
        scratch_types=(
            [pltpu.VMEM((nchunks, K), jnp.int32)]
            + [pltpu.VMEM((K,), jnp.int32)] * 4
            + [pltpu.VMEM((K, d), jnp.float32)] * 2
            + [pltpu.VMEM_SHARED((n_pad, d), jnp.float32)]
            + [pltpu.SemaphoreType.DMA] * 4
        ),
    )
    def scat_kernel(hp_hbm, pk_hbm, out_hbm, *scr):
        ptab = scr[0]
        sbuf = scr[1:3]
        dbuf = scr[3:5]
        bufs = scr[5:7]
        acc_sh = scr[7]
        gsem = scr[8:10]
        ssem = scr[10:12]
        c = lax.axis_index("c")
        s = lax.axis_index("s")
        wid = c * NS + s
        pltpu.sync_copy(pk_hbm.at[wid], ptab)
        zeros = jnp.zeros((L,), jnp.float32)
        cols = d // L

        def zbody(i, carry):
            for c2 in range(cols):
                bufs[0][i, pl.ds(c2 * L, L)] = zeros
            return carry

        lax.fori_loop(0, K, zbody, 0)
        for k2 in range(zpt // K):
            pltpu.sync_copy(bufs[0], acc_sh.at[pl.ds(s * zpt + k2 * K, K)])

        mask = jnp.full((L,), (1 << IDX_BITS) - 1, jnp.int32)
        shift = jnp.full((L,), IDX_BITS, jnp.int32)

        def unpack(j, b):
            for t in range(K // L):
                p = ptab[j, pl.ds(t * L, L)]
                sbuf[b][pl.ds(t * L, L)] = p & mask
                dbuf[b][pl.ds(t * L, L)] = lax.shift_right_logical(p, shift)

        def gather_start(j, b):
            unpack(j, b)
            pltpu.async_copy(hp_hbm.at[sbuf[b]], bufs[b], gsem[b])

        def scatter_start(b):
            pltpu.async_copy(bufs[b], acc_sh.at[dbuf[b]], ssem[b], add=True)

        def wait_chunk(sem, b):
            pltpu.make_async_copy(hp_hbm.at[pl.ds(0, K)], bufs[b], sem).wait()

        plsc.subcore_barrier()

        def body(g, carry):
            for b in range(2):
                j = 2 * g + b

                @pl.when(g > 0)
                def _():
                    wait_chunk(ssem[b], b)
                gather_start(j, b)
                if b == 0:
                    @pl.when(g > 0)
                    def _():
                        wait_chunk(gsem[1], 1)
                        scatter_start(1)
                else:
                    wait_chunk(gsem[0], 0)
                    scatter_start(0)
            return carry

        lax.fori_loop(0, nchunks // 2, body, 0)
        wait_chunk(gsem[1], 1)
        scatter_start(1)
        wait_chunk(ssem[0], 0)
        wait_chunk(ssem[1], 1)
        plsc.subcore_barrier()
        pltpu.sync_copy(acc_sh.at[pl.ds(s * zpt, zpt)],
                        out_hbm.at[c, pl.ds(s * zpt, zpt)])

    return scat_kernel(hp, packed3)


def _dinv_cols(dp_block):
    nw = dp_block.shape[1]
    ones = jnp.ones((nw, 1), jnp.float32)
    deg = jnp.dot(dp_block, ones, preferred_element_type=jnp.float32)
    return lax.rsqrt(deg + 1.0)


def _matmul_call(x, W, deg_t):
    n, d = x.shape
    nw = deg_t.shape[1]
    blk = 2000
    grid = n // blk

    def body(x_ref, w_ref, dp_ref, o_ref):
        dinv = _dinv_cols(dp_ref[...])
        h = jnp.dot(x_ref[...], w_ref[...],
                    preferred_element_type=jnp.float32)
        o_ref[...] = h * dinv

    return pl.pallas_call(
        body,
        grid=(grid,),
        in_specs=[
            pl.BlockSpec((blk, d), lambda i: (i, 0)),
            pl.BlockSpec((d, d), lambda i: (0, 0)),
            pl.BlockSpec((blk, nw), lambda i: (i, 0)),
        ],
        out_specs=pl.BlockSpec((blk, d), lambda i: (i, 0)),
        out_shape=jax.ShapeDtypeStruct((n, d), jnp.float32),
    )(x, W, deg_t)


def _epilogue_call(x, hp, s_parts, deg_t, b2):
    n, d = x.shape
    nw = deg_t.shape[1]
    blk = 2000
    grid = n // blk

    def body(x_ref, hp_ref, s0_ref, s1_ref, dp_ref, b_ref, o_ref):
        dinv = _dinv_cols(dp_ref[...])
        stot = (s0_ref[...].reshape(blk, d) + s1_ref[...].reshape(blk, d)
                + hp_ref[...])
        agg = stot * dinv + b_ref[...]
        o_ref[...] = x_ref[...] + jnp.maximum(agg, 0.0)

    return pl.pallas_call(
        body,
        grid=(grid,),
        in_specs=[
            pl.BlockSpec((blk, d), lambda i: (i, 0)),
            pl.BlockSpec((blk, d), lambda i: (i, 0)),
            pl.BlockSpec((1, blk, d), lambda i: (0, i, 0)),
            pl.BlockSpec((1, blk, d), lambda i: (1, i, 0)),
            pl.BlockSpec((blk, nw), lambda i: (i, 0)),
            pl.BlockSpec((1, d), lambda i: (0, 0)),
        ],
        out_specs=pl.BlockSpec((blk, d), lambda i: (i, 0)),
        out_shape=jax.ShapeDtypeStruct((n, d), jnp.float32),
    )(x, hp, s_parts, s_parts, deg_t, b2)


def kernel(x, edge_index, W, b):
    n, d = x.shape
    e = edge_index.shape[1]
    cpt = NC * NS * K * 2
    e_pad = -(-e // cpt) * cpt
    packed = jnp.bitwise_or(jnp.left_shift(edge_index[1], IDX_BITS),
                            edge_index[0])
    pad = e_pad - e
    if pad:
        pi = np.arange(pad, dtype=np.int64)
        pad_pk = (((n + pi % (_node_pad(n) - n)) << IDX_BITS)
                  | (pi % n)).astype(np.int32)
        packed = jnp.concatenate([packed, jnp.asarray(pad_pk)])

    deg_parts = _deg_call(packed, n)
    deg_t = deg_parts.T
    hp = _matmul_call(x, W, deg_t)
    s_parts = _scatter_call(hp, packed)
    y = _epilogue_call(x, hp, s_parts, deg_t, b.reshape(1, d))
    return y


# speedup vs baseline: 1.1669x; 1.0836x over previous
import functools

import numpy as np

import jax
import jax.numpy as jnp
from jax import lax
from jax.experimental import pallas as pl
from jax.experimental.pallas import tpu as pltpu
from jax.experimental.pallas import tpu_sc as plsc

NC = 2
NS = 16
L = 16
K = 128


def _node_pad(n):
    return -(-(n + 1) // (NS * K)) * (NS * K)


def _deg_call(edge_index, n, e_pad):
    e = edge_index.shape[1]
    nw = NC * NS
    epw = e_pad // nw
    n_pad = _node_pad(n)
    mesh = plsc.VectorSubcoreMesh(core_axis_name="c", subcore_axis_name="s")

    @functools.partial(
        pl.kernel,
        mesh=mesh,
        out_type=jax.ShapeDtypeStruct((nw, n_pad), jnp.float32),
        scratch_types=[
            pltpu.VMEM((epw,), jnp.int32),
            pltpu.VMEM((n_pad,), jnp.float32),
        ],
        compiler_params=pltpu.CompilerParams(needs_layout_passes=False),
    )
    def deg_kernel(ei_hbm, out_hbm, idx_v, deg_v):
        c = lax.axis_index("c")
        s = lax.axis_index("s")
        wid = c * NS + s
        zeros = jnp.zeros((L,), jnp.float32)

        def zbody(i, carry):
            deg_v[pl.ds(i * L, L)] = zeros
            return carry

        lax.fori_loop(0, n_pad // L, zbody, 0)
        ones = jnp.ones((L,), jnp.float32)

        def hbody(i, carry):
            idx = idx_v[pl.ds(i * L, L)]
            plsc.addupdate_scatter(deg_v, [idx], ones)
            return carry

        full_tiles = e // epw
        rem_e = e - full_tiles * epw

        @pl.when(wid < full_tiles)
        def _():
            pltpu.sync_copy(ei_hbm.at[1, pl.ds(wid * epw, epw)], idx_v)
            lax.fori_loop(0, epw // L, hbody, 0)

        if rem_e:
            @pl.when(wid == full_tiles)
            def _():
                pltpu.sync_copy(ei_hbm.at[1, pl.ds(wid * epw, rem_e)],
                                idx_v.at[pl.ds(0, rem_e)])
                lax.fori_loop(0, rem_e // L, hbody, 0)

        pltpu.sync_copy(deg_v, out_hbm.at[wid])

    return deg_kernel(edge_index)


IDX_BITS = 14


def _pack_call(edge_index, pad_pk3, e_pad):
    e = edge_index.shape[1]
    nw = NC * NS
    epw = e_pad // nw
    nchunks = epw // K
    full_tiles = e // epw
    rem_e = e - full_tiles * epw
    mesh = plsc.VectorSubcoreMesh(core_axis_name="c", subcore_axis_name="s")

    @functools.partial(
        pl.kernel,
        mesh=mesh,
        out_type=jax.ShapeDtypeStruct((nw, nchunks, K), jnp.int32),
        scratch_types=[
            pltpu.VMEM((epw,), jnp.int32),
            pltpu.VMEM((epw,), jnp.int32),
            pltpu.VMEM((nchunks, K), jnp.int32),
        ],
    )
    def pack_kernel(ei_hbm, pad_hbm, out_hbm, sidx_v, didx_v, pk_v):
        c = lax.axis_index("c")
        s = lax.axis_index("s")
        wid = c * NS + s
        shift = jnp.full((L,), IDX_BITS, jnp.int32)

        def pbody(r, carry):
            for t in range(K // L):
                o = r * K + t * L
                s_v = sidx_v[pl.ds(o, L)]
                d_v = didx_v[pl.ds(o, L)]
                pk_v[r, pl.ds(t * L, L)] = jnp.bitwise_or(
                    lax.shift_left(d_v, shift), s_v)
            return carry

        @pl.when(wid < full_tiles)
        def _():
            pltpu.sync_copy(ei_hbm.at[0, pl.ds(wid * epw, epw)], sidx_v)
            pltpu.sync_copy(ei_hbm.at[1, pl.ds(wid * epw, epw)], didx_v)
            lax.fori_loop(0, nchunks, pbody, 0)

        if rem_e:
            @pl.when(wid == full_tiles)
            def _():
                pltpu.sync_copy(ei_hbm.at[0, pl.ds(wid * epw, rem_e)],
                                sidx_v.at[pl.ds(0, rem_e)])
                pltpu.sync_copy(ei_hbm.at[1, pl.ds(wid * epw, rem_e)],
                                didx_v.at[pl.ds(0, rem_e)])
                lax.fori_loop(0, rem_e // K, pbody, 0)
                pltpu.sync_copy(pad_hbm,
                                pk_v.at[pl.ds(rem_e // K, pad_pk3.shape[0])])

        pltpu.sync_copy(pk_v, out_hbm.at[wid])

    return pack_kernel(edge_index, pad_pk3)


def _scatter_call(hp, packed3):
    n, d = hp.shape
    nw, nchunks, _ = packed3.shape
    n_pad = _node_pad(n)
    zpt = n_pad // NS
    mesh = plsc.VectorSubcoreMesh(core_axis_name="c", subcore_axis_name="s")

    @functools.partial(
        pl.kernel,
        mesh=mesh,
        out_type=jax.ShapeDtypeStruct((NC, n_pad, d), jnp.float32),
        scratch_types=(
            [pltpu.VMEM((nchunks, K), jnp.int32)]
            + [pltpu.VMEM((K,), jnp.int32)] * 4
            + [pltpu.VMEM((K, d), jnp.float32)] * 2
            + [pltpu.VMEM_SHARED((n_pad, d), jnp.float32)]
            + [pltpu.SemaphoreType.DMA] * 4
        ),
    )
    def scat_kernel(hp_hbm, pk_hbm, out_hbm, *scr):
        ptab = scr[0]
        sbuf = scr[1:3]
        dbuf = scr[3:5]
        bufs = scr[5:7]
        acc_sh = scr[7]
        gsem = scr[8:10]
        ssem = scr[10:12]
        c = lax.axis_index("c")
        s = lax.axis_index("s")
        wid = c * NS + s
        pltpu.sync_copy(pk_hbm.at[wid], ptab)
        zeros = jnp.zeros((L,), jnp.float32)
        cols = d // L

        def zbody(i, carry):
            for c2 in range(cols):
                bufs[0][i, pl.ds(c2 * L, L)] = zeros
            return carry

        lax.fori_loop(0, K, zbody, 0)
        for k2 in range(zpt // K):
            pltpu.sync_copy(bufs[0], acc_sh.at[pl.ds(s * zpt + k2 * K, K)])

        mask = jnp.full((L,), (1 << IDX_BITS) - 1, jnp.int32)
        shift = jnp.full((L,), IDX_BITS, jnp.int32)

        def unpack(j, b):
            for t in range(K // L):
                p = ptab[j, pl.ds(t * L, L)]
                sbuf[b][pl.ds(t * L, L)] = p & mask
                dbuf[b][pl.ds(t * L, L)] = lax.shift_right_logical(p, shift)

        def gather_start(j, b):
            unpack(j, b)
            pltpu.async_copy(hp_hbm.at[sbuf[b]], bufs[b], gsem[b])

        def scatter_start(b):
            pltpu.async_copy(bufs[b], acc_sh.at[dbuf[b]], ssem[b], add=True)

        def wait_chunk(sem, b):
            pltpu.make_async_copy(hp_hbm.at[pl.ds(0, K)], bufs[b], sem).wait()

        plsc.subcore_barrier()

        def body(g, carry):
            for b in range(2):
                j = 2 * g + b

                @pl.when(g > 0)
                def _():
                    wait_chunk(ssem[b], b)
                gather_start(j, b)
                if b == 0:
                    @pl.when(g > 0)
                    def _():
                        wait_chunk(gsem[1], 1)
                        scatter_start(1)
                else:
                    wait_chunk(gsem[0], 0)
                    scatter_start(0)
            return carry

        lax.fori_loop(0, nchunks // 2, body, 0)
        wait_chunk(gsem[1], 1)
        scatter_start(1)
        wait_chunk(ssem[0], 0)
        wait_chunk(ssem[1], 1)
        plsc.subcore_barrier()
        pltpu.sync_copy(acc_sh.at[pl.ds(s * zpt, zpt)],
                        out_hbm.at[c, pl.ds(s * zpt, zpt)])

    return scat_kernel(hp, packed3)


def _dinv_cols(dp_block):
    nw = dp_block.shape[1]
    ones = jnp.ones((nw, 1), jnp.float32)
    deg = jnp.dot(dp_block, ones, preferred_element_type=jnp.float32)
    return lax.rsqrt(deg + 1.0)


def _matmul_call(x, W, deg_t):
    n, d = x.shape
    nw = deg_t.shape[1]
    blk = 2000
    grid = n // blk

    def body(x_ref, w_ref, dp_ref, o_ref):
        dinv = _dinv_cols(dp_ref[...])
        h = jnp.dot(x_ref[...], w_ref[...],
                    preferred_element_type=jnp.float32)
        o_ref[...] = h * dinv

    return pl.pallas_call(
        body,
        grid=(grid,),
        in_specs=[
            pl.BlockSpec((blk, d), lambda i: (i, 0)),
            pl.BlockSpec((d, d), lambda i: (0, 0)),
            pl.BlockSpec((blk, nw), lambda i: (i, 0)),
        ],
        out_specs=pl.BlockSpec((blk, d), lambda i: (i, 0)),
        out_shape=jax.ShapeDtypeStruct((n, d), jnp.float32),
    )(x, W, deg_t)


def _epilogue_call(x, hp, s_parts, deg_t, b2):
    n, d = x.shape
    nw = deg_t.shape[1]
    blk = 2000
    grid = n // blk

    def body(x_ref, hp_ref, s0_ref, s1_ref, dp_ref, b_ref, o_ref):
        dinv = _dinv_cols(dp_ref[...])
        stot = (s0_ref[...].reshape(blk, d) + s1_ref[...].reshape(blk, d)
                + hp_ref[...])
        agg = stot * dinv + b_ref[...]
        o_ref[...] = x_ref[...] + jnp.maximum(agg, 0.0)

    return pl.pallas_call(
        body,
        grid=(grid,),
        in_specs=[
            pl.BlockSpec((blk, d), lambda i: (i, 0)),
            pl.BlockSpec((blk, d), lambda i: (i, 0)),
            pl.BlockSpec((1, blk, d), lambda i: (0, i, 0)),
            pl.BlockSpec((1, blk, d), lambda i: (1, i, 0)),
            pl.BlockSpec((blk, nw), lambda i: (i, 0)),
            pl.BlockSpec((1, d), lambda i: (0, 0)),
        ],
        out_specs=pl.BlockSpec((blk, d), lambda i: (i, 0)),
        out_shape=jax.ShapeDtypeStruct((n, d), jnp.float32),
    )(x, hp, s_parts, s_parts, deg_t, b2)


def kernel(x, edge_index, W, b):
    n, d = x.shape
    e = edge_index.shape[1]
    cpt = NC * NS * K * 2
    e_pad = -(-e // cpt) * cpt
    pad = e_pad - e
    pi = np.arange(pad, dtype=np.int64)
    pad_pk = (((n + pi % (_node_pad(n) - n)) << IDX_BITS)
              | (pi % n)).astype(np.int32)
    pad_pk3 = jnp.asarray(pad_pk.reshape(pad // K, K))

    deg_parts = _deg_call(edge_index, n, e_pad)
    packed3 = _pack_call(edge_index, pad_pk3, e_pad)
    deg_t = deg_parts.T
    hp = _matmul_call(x, W, deg_t)
    s_parts = _scatter_call(hp, packed3)
    y = _epilogue_call(x, hp, s_parts, deg_t, b.reshape(1, d))
    return y
